# baseline (speedup 1.0000x reference)
import jax
import jax.numpy as jnp
from jax.experimental import pallas as pl
from jax.experimental.pallas import tpu as pltpu

HS = 501
HSP = 512
N = 8
NVT = 10
NVTP = 16
NZ = 56


def _pad_to(x, shape):
    return jnp.pad(x, [(0, s - d) for s, d in zip(shape, x.shape)])


def _body(x_ref, w_ref, m_ref, wih_ref, bih_ref, whh_ref, bhh_ref,
          wgm_ref, bg_ref, vid_ref, w12_ref, b12_ref, out_ref, h_scr):
    f32 = jnp.float32
    BB = x_ref.shape[0]
    NH = 2
    HB = BB // NH

    def dot(a, b):
        return jax.lax.dot_general(a, b, (((1,), (0,)), ((), ())))

    for v in range(N):
        hvs = []
        for h in range(NH):
            rs = pl.ds(h * HB, HB)
            xv = x_ref[rs, v * NVTP:(v + 1) * NVTP]
            gi = dot(xv, wih_ref[...]) + bih_ref[0:1, :]
            if v == 0:
                hin = jnp.zeros((HB, HSP), f32)
                gh = jnp.zeros((HB, 3 * HSP), f32) + bhh_ref[0:1, :]
            else:
                hin = jnp.zeros((HB, HSP), f32)
                for x in range(v):
                    idx = x * N + v
                    ws = w_ref[rs, idx:idx + 1]
                    m = m_ref[rs, idx:idx + 1]
                    msg = m * (ws * h_scr[x, rs, :] + vid_ref[x:x + 1, :])
                    gm = dot(msg, wgm_ref[...])
                    gate = jax.nn.sigmoid(gm[:, :HSP] + bg_ref[0:1, :])
                    hin = hin + gate * gm[:, HSP:]
                gh = dot(hin, whh_ref[...]) + bhh_ref[0:1, :]
            r = jax.nn.sigmoid(gi[:, :HSP] + gh[:, :HSP])
            z = jax.nn.sigmoid(gi[:, HSP:2 * HSP] + gh[:, HSP:2 * HSP])
            n = jnp.tanh(gi[:, 2 * HSP:] + r * gh[:, 2 * HSP:])
            hv = (1.0 - z) * n + z * hin
            hvs.append(hv)
        if v < N - 1:
            for h in range(NH):
                h_scr[v, pl.ds(h * HB, HB), :] = hvs[h]
        else:
            for h in range(NH):
                out_ref[pl.ds(h * HB, HB), :] = (
                    dot(hvs[h], w12_ref[...]) + b12_ref[0:1, :])


def kernel(node_types, edge_type, adj_mask, W_ih, W_hh, b_ih, b_hh,
           Wg, bg, Wm, W1, b1, W2, b2):
    f32 = jnp.float32
    B = node_types.shape[0]

    def split3_t(W, kpad):
        parts = [_pad_to(W[i * HS:(i + 1) * HS].T, (W.shape[1], HSP))
                 for i in range(3)]
        return _pad_to(jnp.concatenate(parts, axis=1), (kpad, 3 * HSP))

    wih = split3_t(W_ih, NVTP)
    whh = split3_t(W_hh, HSP)

    def bias3(b):
        return jnp.concatenate(
            [_pad_to(b[i * HS:(i + 1) * HS], (HSP,)) for i in range(3)])[None]

    bih = bias3(b_ih)
    bhh = bias3(b_hh)

    wgm = jnp.concatenate([_pad_to(Wg.T, (HSP, HSP)),
                           _pad_to(Wm.T, (HSP, HSP))], axis=1)
    bgp = _pad_to(bg, (HSP,))[None]
    vid = _pad_to(jnp.concatenate(
        [jnp.zeros((N, HS), f32), jnp.eye(N, dtype=f32)], axis=1),
        (N, HSP))

    w12 = _pad_to(jnp.concatenate([W1.T, W2.T], axis=1), (HSP, 2 * NZ))
    b12 = jnp.concatenate([b1, b2])[None]

    X = jax.nn.one_hot(node_types, NVT, dtype=f32)
    X = _pad_to(X, (B, N, NVTP)).reshape(B, N * NVTP)
    dag = jnp.triu(jnp.ones((N, N), f32), k=1)[None]
    msk = (adj_mask.astype(f32) * dag).reshape(B, N * N)
    wsc = (edge_type.astype(f32) * 10.0 + 1.0).reshape(B, N * N)

    BB = 256
    nblk = B // BB
    const = lambda i: (0, 0)
    blk = lambda i: (i, 0)
    out = pl.pallas_call(
        _body,
        grid=(nblk,),
        in_specs=[
            pl.BlockSpec((BB, N * NVTP), blk),
            pl.BlockSpec((BB, N * N), blk),
            pl.BlockSpec((BB, N * N), blk),
            pl.BlockSpec((NVTP, 3 * HSP), const),
            pl.BlockSpec((1, 3 * HSP), const),
            pl.BlockSpec((HSP, 3 * HSP), const),
            pl.BlockSpec((1, 3 * HSP), const),
            pl.BlockSpec((HSP, 2 * HSP), const),
            pl.BlockSpec((1, HSP), const),
            pl.BlockSpec((N, HSP), const),
            pl.BlockSpec((HSP, 2 * NZ), const),
            pl.BlockSpec((1, 2 * NZ), const),
        ],
        out_specs=pl.BlockSpec((BB, 2 * NZ), blk),
        out_shape=jax.ShapeDtypeStruct((B, 2 * NZ), f32),
        scratch_shapes=[pltpu.VMEM((N - 1, BB, HSP), f32)],
    )(X, wsc, msk, wih, bih, whh, bhh, wgm, bgp, vid, w12, b12)
    return out[:, :NZ], out[:, NZ:]


# speedup vs baseline: 1.0939x; 1.0939x over previous
import jax
import jax.numpy as jnp
from jax.experimental import pallas as pl
from jax.experimental.pallas import tpu as pltpu

HS = 501
HSP = 512
N = 8
NVT = 10
NVTP = 16
NZ = 56


def _pad_to(x, shape):
    return jnp.pad(x, [(0, s - d) for s, d in zip(shape, x.shape)])


def _body(x_ref, w_ref, m_ref, wih_ref, bih_ref, whh_ref, bhh_ref,
          wgm_ref, bg_ref, vid_ref, w12_ref, b12_ref, out_ref, h_scr):
    f32 = jnp.float32
    BB = x_ref.shape[0]

    def dot(a, b):
        return jax.lax.dot_general(a, b, (((1,), (1,)), ((), ())))

    for v in range(N):
        xv = x_ref[:, v * NVTP:(v + 1) * NVTP]
        gi = dot(xv, wih_ref[...]) + bih_ref[0:1, :]
        if v == 0:
            hin = jnp.zeros((BB, HSP), f32)
            gh = jnp.zeros((BB, 3 * HSP), f32) + bhh_ref[0:1, :]
        else:
            hin = jnp.zeros((BB, HSP), f32)
            for x in range(v):
                idx = x * N + v
                ws = w_ref[:, idx:idx + 1]
                m = m_ref[:, idx:idx + 1]
                msg = m * (ws * h_scr[x] + vid_ref[x:x + 1, :])
                gm = dot(msg, wgm_ref[...])
                gate = jax.nn.sigmoid(gm[:, :HSP] + bg_ref[0:1, :])
                hin = hin + gate * gm[:, HSP:]
            gh = dot(hin, whh_ref[...]) + bhh_ref[0:1, :]
        r = jax.nn.sigmoid(gi[:, :HSP] + gh[:, :HSP])
        z = jax.nn.sigmoid(gi[:, HSP:2 * HSP] + gh[:, HSP:2 * HSP])
        n = jnp.tanh(gi[:, 2 * HSP:] + r * gh[:, 2 * HSP:])
        hv = (1.0 - z) * n + z * hin
        if v < N - 1:
            h_scr[v] = hv
        else:
            out_ref[...] = dot(hv, w12_ref[...]) + b12_ref[0:1, :]


def kernel(node_types, edge_type, adj_mask, W_ih, W_hh, b_ih, b_hh,
           Wg, bg, Wm, W1, b1, W2, b2):
    f32 = jnp.float32
    B = node_types.shape[0]

    def split3(W, kpad):
        return jnp.concatenate(
            [_pad_to(W[i * HS:(i + 1) * HS], (HSP, kpad)) for i in range(3)])

    wih = split3(W_ih, NVTP)
    whh = split3(W_hh, HSP)

    def bias3(b):
        return jnp.concatenate(
            [_pad_to(b[i * HS:(i + 1) * HS], (HSP,)) for i in range(3)])[None]

    bih = bias3(b_ih)
    bhh = bias3(b_hh)

    wgm = jnp.concatenate([_pad_to(Wg, (HSP, HSP)),
                           _pad_to(Wm, (HSP, HSP))])
    bgp = _pad_to(bg, (HSP,))[None]
    vid = _pad_to(jnp.concatenate(
        [jnp.zeros((N, HS), f32), jnp.eye(N, dtype=f32)], axis=1),
        (N, HSP))

    w12 = _pad_to(jnp.concatenate([W1, W2]), (2 * NZ, HSP))
    b12 = jnp.concatenate([b1, b2])[None]

    X = jax.nn.one_hot(node_types, NVT, dtype=f32)
    X = _pad_to(X, (B, N, NVTP)).reshape(B, N * NVTP)
    dag = jnp.triu(jnp.ones((N, N), f32), k=1)[None]
    msk = (adj_mask.astype(f32) * dag).reshape(B, N * N)
    wsc = (edge_type.astype(f32) * 10.0 + 1.0).reshape(B, N * N)

    BB = 256
    nblk = B // BB
    const = lambda i: (0, 0)
    blk = lambda i: (i, 0)
    out = pl.pallas_call(
        _body,
        grid=(nblk,),
        in_specs=[
            pl.BlockSpec((BB, N * NVTP), blk),
            pl.BlockSpec((BB, N * N), blk),
            pl.BlockSpec((BB, N * N), blk),
            pl.BlockSpec((3 * HSP, NVTP), const),
            pl.BlockSpec((1, 3 * HSP), const),
            pl.BlockSpec((3 * HSP, HSP), const),
            pl.BlockSpec((1, 3 * HSP), const),
            pl.BlockSpec((2 * HSP, HSP), const),
            pl.BlockSpec((1, HSP), const),
            pl.BlockSpec((N, HSP), const),
            pl.BlockSpec((2 * NZ, HSP), const),
            pl.BlockSpec((1, 2 * NZ), const),
        ],
        out_specs=pl.BlockSpec((BB, 2 * NZ), blk),
        out_shape=jax.ShapeDtypeStruct((B, 2 * NZ), f32),
        scratch_shapes=[pltpu.VMEM((N - 1, BB, HSP), f32)],
    )(X, wsc, msk, wih, bih, whh, bhh, wgm, bgp, vid, w12, b12)
    return out[:, :NZ], out[:, NZ:]


# speedup vs baseline: 1.1475x; 1.0489x over previous
import jax
import jax.numpy as jnp
from jax.experimental import pallas as pl
from jax.experimental.pallas import tpu as pltpu

HS = 501
HSP = 512
N = 8
NVT = 10
NVTP = 16
NZ = 56


def _pad_to(x, shape):
    return jnp.pad(x, [(0, s - d) for s, d in zip(shape, x.shape)])


def _body(x_ref, w_ref, m_ref, wih_ref, bih_ref, whh_ref, bhh_ref,
          wgm_ref, bg_ref, vid_ref, w12_ref, b12_ref, out_ref, h_scr):
    f32 = jnp.float32
    BB = x_ref.shape[0]

    def dot(a, b):
        return jax.lax.dot_general(a.astype(jnp.bfloat16), b,
                                   (((1,), (1,)), ((), ())),
                                   preferred_element_type=f32)

    for v in range(N):
        xv = x_ref[:, v * NVTP:(v + 1) * NVTP]
        gi = dot(xv, wih_ref[...]) + bih_ref[0:1, :]
        if v == 0:
            hin = jnp.zeros((BB, HSP), f32)
            gh = jnp.zeros((BB, 3 * HSP), f32) + bhh_ref[0:1, :]
        else:
            hin = jnp.zeros((BB, HSP), f32)
            for x in range(v):
                idx = x * N + v
                ws = w_ref[:, idx:idx + 1]
                m = m_ref[:, idx:idx + 1]
                msg = m * (ws * h_scr[x] + vid_ref[x:x + 1, :])
                gm = dot(msg, wgm_ref[...])
                gate = jax.nn.sigmoid(gm[:, :HSP] + bg_ref[0:1, :])
                hin = hin + gate * gm[:, HSP:]
            gh = dot(hin, whh_ref[...]) + bhh_ref[0:1, :]
        r = jax.nn.sigmoid(gi[:, :HSP] + gh[:, :HSP])
        z = jax.nn.sigmoid(gi[:, HSP:2 * HSP] + gh[:, HSP:2 * HSP])
        n = jnp.tanh(gi[:, 2 * HSP:] + r * gh[:, 2 * HSP:])
        hv = (1.0 - z) * n + z * hin
        if v < N - 1:
            h_scr[v] = hv
        else:
            out_ref[...] = dot(hv, w12_ref[...]) + b12_ref[0:1, :]


def kernel(node_types, edge_type, adj_mask, W_ih, W_hh, b_ih, b_hh,
           Wg, bg, Wm, W1, b1, W2, b2):
    f32 = jnp.float32
    B = node_types.shape[0]

    def split3(W, kpad):
        return jnp.concatenate(
            [_pad_to(W[i * HS:(i + 1) * HS], (HSP, kpad)) for i in range(3)])

    bf16 = jnp.bfloat16
    wih = split3(W_ih, NVTP).astype(bf16)
    whh = split3(W_hh, HSP).astype(bf16)

    def bias3(b):
        return jnp.concatenate(
            [_pad_to(b[i * HS:(i + 1) * HS], (HSP,)) for i in range(3)])[None]

    bih = bias3(b_ih)
    bhh = bias3(b_hh)

    wgm = jnp.concatenate([_pad_to(Wg, (HSP, HSP)),
                           _pad_to(Wm, (HSP, HSP))]).astype(bf16)
    bgp = _pad_to(bg, (HSP,))[None]
    vid = _pad_to(jnp.concatenate(
        [jnp.zeros((N, HS), f32), jnp.eye(N, dtype=f32)], axis=1),
        (N, HSP))

    w12 = _pad_to(jnp.concatenate([W1, W2]), (2 * NZ, HSP)).astype(bf16)
    b12 = jnp.concatenate([b1, b2])[None]

    X = jax.nn.one_hot(node_types, NVT, dtype=f32)
    X = _pad_to(X, (B, N, NVTP)).reshape(B, N * NVTP)
    dag = jnp.triu(jnp.ones((N, N), f32), k=1)[None]
    msk = (adj_mask.astype(f32) * dag).reshape(B, N * N)
    wsc = (edge_type.astype(f32) * 10.0 + 1.0).reshape(B, N * N)

    BB = 256
    nblk = B // BB
    const = lambda i: (0, 0)
    blk = lambda i: (i, 0)
    out = pl.pallas_call(
        _body,
        grid=(nblk,),
        in_specs=[
            pl.BlockSpec((BB, N * NVTP), blk),
            pl.BlockSpec((BB, N * N), blk),
            pl.BlockSpec((BB, N * N), blk),
            pl.BlockSpec((3 * HSP, NVTP), const),
            pl.BlockSpec((1, 3 * HSP), const),
            pl.BlockSpec((3 * HSP, HSP), const),
            pl.BlockSpec((1, 3 * HSP), const),
            pl.BlockSpec((2 * HSP, HSP), const),
            pl.BlockSpec((1, HSP), const),
            pl.BlockSpec((N, HSP), const),
            pl.BlockSpec((2 * NZ, HSP), const),
            pl.BlockSpec((1, 2 * NZ), const),
        ],
        out_specs=pl.BlockSpec((BB, 2 * NZ), blk),
        out_shape=jax.ShapeDtypeStruct((B, 2 * NZ), f32),
        scratch_shapes=[pltpu.VMEM((N - 1, BB, HSP), f32)],
    )(X, wsc, msk, wih, bih, whh, bhh, wgm, bgp, vid, w12, b12)
    return out[:, :NZ], out[:, NZ:]
